# initial kernel scaffold (unmeasured)
import jax
import jax.numpy as jnp
from jax import lax
from jax.experimental import pallas as pl
from jax.experimental.pallas import tpu as pltpu

N_DEV = 8
N_CHUNKS = 4


def kernel(x, w_mat):
    m, k_sh = x.shape
    _, n = w_mat.shape
    mb = m // N_DEV
    nc = n // N_CHUNKS

    def body(x_ref, w_ref, out_ref, comm, amax_buf,
             send_sems, recv_sems, amax_send_sems, amax_recv_sems,
             credit_sem, out_sem):
        p = lax.axis_index("i")
        left = lax.rem(p - 1 + N_DEV, N_DEV)
        right = lax.rem(p + 1, N_DEV)

        barrier_sem = pltpu.get_barrier_semaphore()
        for nbr in (left, right):
            pl.semaphore_signal(
                barrier_sem, 1,
                device_id=(nbr,), device_id_type=pl.DeviceIdType.MESH,
            )
        pl.semaphore_wait(barrier_sem, 2)

        def accum_partial(slot, b, add):
            xs = x_ref[pl.ds(b * mb, mb), :]
            for j in range(N_CHUNKS):
                cols = pl.ds(j * nc, nc)
                pb = jnp.dot(
                    xs, w_ref[:, cols],
                    precision=lax.Precision.HIGHEST,
                    preferred_element_type=jnp.float32,
                )
                if add:
                    comm[slot, :, cols] = comm[slot, :, cols] + pb
                else:
                    comm[slot, :, cols] = pb

        accum_partial(0, lax.rem(p - 1 + N_DEV, N_DEV), add=False)

        for s in range(N_DEV - 1):
            if s >= 1:
                pl.semaphore_wait(credit_sem, 1)
            rdma = pltpu.make_async_remote_copy(
                src_ref=comm.at[s % 2],
                dst_ref=comm.at[(s + 1) % 2],
                send_sem=send_sems.at[s % 2],
                recv_sem=recv_sems.at[(s + 1) % 2],
                device_id=(right,),
                device_id_type=pl.DeviceIdType.MESH,
            )
            rdma.start()
            rdma.wait()
            if s < N_DEV - 2:
                pl.semaphore_signal(
                    credit_sem, 1,
                    device_id=(left,), device_id_type=pl.DeviceIdType.MESH,
                )
            b = lax.rem(p - s - 2 + 2 * N_DEV, N_DEV)
            if s < N_DEV - 2:
                accum_partial((s + 1) % 2, b, add=True)
            else:
                xs = x_ref[pl.ds(b * mb, mb), :]
                for j in range(N_CHUNKS):
                    cols = pl.ds(j * nc, nc)
                    pb = jnp.dot(
                        xs, w_ref[:, cols],
                        precision=lax.Precision.HIGHEST,
                        preferred_element_type=jnp.float32,
                    )
                    comm[0, :, cols] = comm[1, :, cols] + pb

        amax_local = jnp.float32(0.0)
        for j in range(N_CHUNKS):
            cols = pl.ds(j * nc, nc)
            amax_local = jnp.maximum(
                amax_local, jnp.max(jnp.abs(comm[0, :, cols]))
            )
        amax_buf[pl.ds(0, 1), :] = jnp.full((1, 128), amax_local, jnp.float32)

        rds = []
        for d in range(1, N_DEV):
            tgt = lax.rem(p + d, N_DEV)
            rd = pltpu.make_async_remote_copy(
                src_ref=amax_buf.at[pl.ds(0, 1)],
                dst_ref=amax_buf.at[pl.ds(d, 1)],
                send_sem=amax_send_sems.at[d],
                recv_sem=amax_recv_sems.at[d],
                device_id=(tgt,),
                device_id_type=pl.DeviceIdType.MESH,
            )
            rd.start()
            rds.append(rd)
        for rd in rds:
            rd.wait_send()
        for rd in rds:
            rd.wait_recv()
        gmax = jnp.max(amax_buf[:, :])

        scale = gmax / 448.0
        inv_scale = 448.0 / gmax
        for j in range(N_CHUNKS):
            cols = pl.ds(j * nc, nc)
            q = jnp.clip(
                comm[0, :, cols] * inv_scale, -448.0, 448.0
            ).astype(jnp.float8_e4m3fn)
            comm[0, :, cols] = q.astype(jnp.float32) * scale

        cp = pltpu.make_async_copy(comm.at[0], out_ref, out_sem)
        cp.start()
        cp.wait()

    return pl.pallas_call(
        body,
        out_shape=jax.ShapeDtypeStruct((mb, n), jnp.float32),
        in_specs=[
            pl.BlockSpec(memory_space=pltpu.MemorySpace.VMEM),
            pl.BlockSpec(memory_space=pltpu.MemorySpace.VMEM),
        ],
        out_specs=pl.BlockSpec(memory_space=pltpu.MemorySpace.HBM),
        scratch_shapes=[
            pltpu.VMEM((2, mb, n), jnp.float32),
            pltpu.VMEM((N_DEV, 128), jnp.float32),
            pltpu.SemaphoreType.DMA((2,)),
            pltpu.SemaphoreType.DMA((2,)),
            pltpu.SemaphoreType.DMA((N_DEV,)),
            pltpu.SemaphoreType.DMA((N_DEV,)),
            pltpu.SemaphoreType.REGULAR,
            pltpu.SemaphoreType.DMA,
        ],
        compiler_params=pltpu.CompilerParams(collective_id=0),
    )(x, w_mat)


# baseline (device time: 1423999 ns/iter reference)
import jax
import jax.numpy as jnp
from jax import lax
from jax.experimental import pallas as pl
from jax.experimental.pallas import tpu as pltpu

N_DEV = 8
N_CHUNKS = 4


def kernel(x, w_mat):
    m, k_sh = x.shape
    _, n = w_mat.shape
    mb = m // N_DEV
    nc = n // N_CHUNKS

    def body(x_ref, w_ref, out_ref, comm, amax_buf,
             send_sems, recv_sems, amax_send_sems, amax_recv_sems,
             credit_sem, out_sem):
        p = lax.axis_index("i")
        left = lax.rem(p - 1 + N_DEV, N_DEV)
        right = lax.rem(p + 1, N_DEV)

        barrier_sem = pltpu.get_barrier_semaphore()
        for nbr in (left, right):
            pl.semaphore_signal(
                barrier_sem, 1,
                device_id=(nbr,), device_id_type=pl.DeviceIdType.MESH,
            )
        pl.semaphore_wait(barrier_sem, 2)

        def dot3(xs_hi, xs_lo, cols):
            wc = w_ref[:, cols]
            w_hi = wc.astype(jnp.bfloat16)
            w_lo = (wc - w_hi.astype(jnp.float32)).astype(jnp.bfloat16)
            acc = jnp.dot(xs_hi, w_hi, preferred_element_type=jnp.float32)
            acc += jnp.dot(xs_hi, w_lo, preferred_element_type=jnp.float32)
            acc += jnp.dot(xs_lo, w_hi, preferred_element_type=jnp.float32)
            return acc

        def split_x(b):
            xs = x_ref[pl.ds(b * mb, mb), :]
            xs_hi = xs.astype(jnp.bfloat16)
            xs_lo = (xs - xs_hi.astype(jnp.float32)).astype(jnp.bfloat16)
            return xs_hi, xs_lo

        def accum_partial(slot, b, add):
            xs_hi, xs_lo = split_x(b)
            for j in range(N_CHUNKS):
                cols = pl.ds(j * nc, nc)
                pb = dot3(xs_hi, xs_lo, cols)
                if add:
                    comm[slot, :, cols] = comm[slot, :, cols] + pb
                else:
                    comm[slot, :, cols] = pb

        accum_partial(0, lax.rem(p - 1 + N_DEV, N_DEV), add=False)

        for s in range(N_DEV - 1):
            if s >= 1:
                pl.semaphore_wait(credit_sem, 1)
            rdma = pltpu.make_async_remote_copy(
                src_ref=comm.at[s % 2],
                dst_ref=comm.at[(s + 1) % 2],
                send_sem=send_sems.at[s % 2],
                recv_sem=recv_sems.at[(s + 1) % 2],
                device_id=(right,),
                device_id_type=pl.DeviceIdType.MESH,
            )
            rdma.start()
            rdma.wait()
            if s < N_DEV - 2:
                pl.semaphore_signal(
                    credit_sem, 1,
                    device_id=(left,), device_id_type=pl.DeviceIdType.MESH,
                )
            b = lax.rem(p - s - 2 + 2 * N_DEV, N_DEV)
            if s < N_DEV - 2:
                accum_partial((s + 1) % 2, b, add=True)
            else:
                xs_hi, xs_lo = split_x(b)
                for j in range(N_CHUNKS):
                    cols = pl.ds(j * nc, nc)
                    pb = dot3(xs_hi, xs_lo, cols)
                    comm[0, :, cols] = comm[1, :, cols] + pb

        amax_local = jnp.float32(0.0)
        for j in range(N_CHUNKS):
            cols = pl.ds(j * nc, nc)
            amax_local = jnp.maximum(
                amax_local, jnp.max(jnp.abs(comm[0, :, cols]))
            )
        amax_buf[pl.ds(0, 1), :] = jnp.full((1, 128), amax_local, jnp.float32)

        rds = []
        for d in range(1, N_DEV):
            tgt = lax.rem(p + d, N_DEV)
            rd = pltpu.make_async_remote_copy(
                src_ref=amax_buf.at[pl.ds(0, 1)],
                dst_ref=amax_buf.at[pl.ds(d, 1)],
                send_sem=amax_send_sems.at[d],
                recv_sem=amax_recv_sems.at[d],
                device_id=(tgt,),
                device_id_type=pl.DeviceIdType.MESH,
            )
            rd.start()
            rds.append(rd)
        for rd in rds:
            rd.wait_send()
        for rd in rds:
            rd.wait_recv()
        gmax = jnp.max(amax_buf[:, :])

        scale = gmax / 448.0
        inv_scale = 448.0 / gmax
        for j in range(N_CHUNKS):
            cols = pl.ds(j * nc, nc)
            q = jnp.clip(
                comm[0, :, cols] * inv_scale, -448.0, 448.0
            ).astype(jnp.float8_e4m3fn)
            comm[0, :, cols] = q.astype(jnp.float32) * scale

        cp = pltpu.make_async_copy(comm.at[0], out_ref, out_sem)
        cp.start()
        cp.wait()

    return pl.pallas_call(
        body,
        out_shape=jax.ShapeDtypeStruct((mb, n), jnp.float32),
        in_specs=[
            pl.BlockSpec(memory_space=pltpu.MemorySpace.VMEM),
            pl.BlockSpec(memory_space=pltpu.MemorySpace.VMEM),
        ],
        out_specs=pl.BlockSpec(memory_space=pltpu.MemorySpace.HBM),
        scratch_shapes=[
            pltpu.VMEM((2, mb, n), jnp.float32),
            pltpu.VMEM((N_DEV, 128), jnp.float32),
            pltpu.SemaphoreType.DMA((2,)),
            pltpu.SemaphoreType.DMA((2,)),
            pltpu.SemaphoreType.DMA((N_DEV,)),
            pltpu.SemaphoreType.DMA((N_DEV,)),
            pltpu.SemaphoreType.REGULAR,
            pltpu.SemaphoreType.DMA,
        ],
        compiler_params=pltpu.CompilerParams(
            collective_id=0,
            vmem_limit_bytes=100 * 1024 * 1024,
        ),
    )(x, w_mat)


# device time: 797064 ns/iter; 1.7866x vs baseline; 1.7866x over previous
import jax
import jax.numpy as jnp
from jax import lax
from jax.experimental import pallas as pl
from jax.experimental.pallas import tpu as pltpu

N_DEV = 8
N_CHUNKS_H = 2


def kernel(x, w_mat):
    m, k_sh = x.shape
    _, n = w_mat.shape
    mb = m // N_DEV
    nh = n // 2
    nc = nh // N_CHUNKS_H

    def body(x_ref, w_ref, out_ref, comm_f, comm_r, amax_buf,
             send_f, recv_f, send_r, recv_r,
             amax_send_sems, amax_recv_sems,
             credit_f, credit_r, out_sems):
        p = lax.axis_index("i")
        left = lax.rem(p - 1 + N_DEV, N_DEV)
        right = lax.rem(p + 1, N_DEV)

        barrier_sem = pltpu.get_barrier_semaphore()
        for nbr in (left, right):
            pl.semaphore_signal(
                barrier_sem, 1,
                device_id=(nbr,), device_id_type=pl.DeviceIdType.MESH,
            )
        pl.semaphore_wait(barrier_sem, 2)

        def split_x(b):
            xs = x_ref[pl.ds(b * mb, mb), :]
            xs_hi = xs.astype(jnp.bfloat16)
            xs_lo = (xs - xs_hi.astype(jnp.float32)).astype(jnp.bfloat16)
            return xs_hi, xs_lo

        def dot3(xsplit, col_lo):
            xs_hi, xs_lo = xsplit
            wc = w_ref[:, pl.ds(col_lo, nc)]
            w_hi = wc.astype(jnp.bfloat16)
            w_lo = (wc - w_hi.astype(jnp.float32)).astype(jnp.bfloat16)
            acc = jnp.dot(xs_hi, w_hi, preferred_element_type=jnp.float32)
            acc += jnp.dot(xs_hi, w_lo, preferred_element_type=jnp.float32)
            acc += jnp.dot(xs_lo, w_hi, preferred_element_type=jnp.float32)
            return acc

        def accum_half(comm, slot, b, half, add, recv_slot=None):
            xsplit = split_x(b)
            for j in range(N_CHUNKS_H):
                pb = dot3(xsplit, half * nh + j * nc)
                cols = pl.ds(j * nc, nc)
                if add:
                    src = slot if recv_slot is None else recv_slot
                    comm[slot, :, cols] = comm[src, :, cols] + pb
                else:
                    comm[slot, :, cols] = pb

        accum_half(comm_f, 0, lax.rem(p - 1 + N_DEV, N_DEV), 0, add=False)
        accum_half(comm_r, 0, lax.rem(p + 1, N_DEV), 1, add=False)

        for s in range(N_DEV - 1):
            if s >= 1:
                pl.semaphore_wait(credit_f, 1)
                pl.semaphore_wait(credit_r, 1)
            rdma_f = pltpu.make_async_remote_copy(
                src_ref=comm_f.at[s % 2],
                dst_ref=comm_f.at[(s + 1) % 2],
                send_sem=send_f.at[s % 2],
                recv_sem=recv_f.at[(s + 1) % 2],
                device_id=(right,),
                device_id_type=pl.DeviceIdType.MESH,
            )
            rdma_r = pltpu.make_async_remote_copy(
                src_ref=comm_r.at[s % 2],
                dst_ref=comm_r.at[(s + 1) % 2],
                send_sem=send_r.at[s % 2],
                recv_sem=recv_r.at[(s + 1) % 2],
                device_id=(left,),
                device_id_type=pl.DeviceIdType.MESH,
            )
            rdma_f.start()
            rdma_r.start()
            rdma_f.wait()
            rdma_r.wait()
            if s < N_DEV - 2:
                pl.semaphore_signal(
                    credit_f, 1,
                    device_id=(left,), device_id_type=pl.DeviceIdType.MESH,
                )
                pl.semaphore_signal(
                    credit_r, 1,
                    device_id=(right,), device_id_type=pl.DeviceIdType.MESH,
                )
            bf = lax.rem(p - s - 2 + 2 * N_DEV, N_DEV)
            br = lax.rem(p + s + 2, N_DEV)
            if s < N_DEV - 2:
                accum_half(comm_f, (s + 1) % 2, bf, 0, add=True)
                accum_half(comm_r, (s + 1) % 2, br, 1, add=True)
            else:
                accum_half(comm_f, 0, bf, 0, add=True, recv_slot=1)
                accum_half(comm_r, 0, br, 1, add=True, recv_slot=1)

        amax_local = jnp.float32(0.0)
        for comm in (comm_f, comm_r):
            for j in range(N_CHUNKS_H):
                cols = pl.ds(j * nc, nc)
                amax_local = jnp.maximum(
                    amax_local, jnp.max(jnp.abs(comm[0, :, cols]))
                )
        amax_buf[pl.ds(0, 1), :] = jnp.full((1, 128), amax_local, jnp.float32)

        rds = []
        for d in range(1, N_DEV):
            tgt = lax.rem(p + d, N_DEV)
            rd = pltpu.make_async_remote_copy(
                src_ref=amax_buf.at[pl.ds(0, 1)],
                dst_ref=amax_buf.at[pl.ds(d, 1)],
                send_sem=amax_send_sems.at[d],
                recv_sem=amax_recv_sems.at[d],
                device_id=(tgt,),
                device_id_type=pl.DeviceIdType.MESH,
            )
            rd.start()
            rds.append(rd)
        for rd in rds:
            rd.wait_send()
        for rd in rds:
            rd.wait_recv()
        gmax = jnp.max(amax_buf[:, :])

        scale = gmax / 448.0
        inv_scale = 448.0 / gmax
        cps = []
        for comm, half in ((comm_f, 0), (comm_r, 1)):
            for j in range(N_CHUNKS_H):
                cols = pl.ds(j * nc, nc)
                q = jnp.clip(
                    comm[0, :, cols] * inv_scale, -448.0, 448.0
                ).astype(jnp.float8_e4m3fn)
                comm[0, :, cols] = q.astype(jnp.float32) * scale
            cp = pltpu.make_async_copy(
                comm.at[0],
                out_ref.at[:, pl.ds(half * nh, nh)],
                out_sems.at[half],
            )
            cp.start()
            cps.append(cp)
        for cp in cps:
            cp.wait()

    return pl.pallas_call(
        body,
        out_shape=jax.ShapeDtypeStruct((mb, n), jnp.float32),
        in_specs=[
            pl.BlockSpec(memory_space=pltpu.MemorySpace.VMEM),
            pl.BlockSpec(memory_space=pltpu.MemorySpace.VMEM),
        ],
        out_specs=pl.BlockSpec(memory_space=pltpu.MemorySpace.HBM),
        scratch_shapes=[
            pltpu.VMEM((2, mb, nh), jnp.float32),
            pltpu.VMEM((2, mb, nh), jnp.float32),
            pltpu.VMEM((N_DEV, 128), jnp.float32),
            pltpu.SemaphoreType.DMA((2,)),
            pltpu.SemaphoreType.DMA((2,)),
            pltpu.SemaphoreType.DMA((2,)),
            pltpu.SemaphoreType.DMA((2,)),
            pltpu.SemaphoreType.DMA((N_DEV,)),
            pltpu.SemaphoreType.DMA((N_DEV,)),
            pltpu.SemaphoreType.REGULAR,
            pltpu.SemaphoreType.REGULAR,
            pltpu.SemaphoreType.DMA((2,)),
        ],
        compiler_params=pltpu.CompilerParams(
            collective_id=0,
            vmem_limit_bytes=100 * 1024 * 1024,
        ),
    )(x, w_mat)


# device time: 716288 ns/iter; 1.9880x vs baseline; 1.1128x over previous
import jax
import jax.numpy as jnp
from jax import lax
from jax.experimental import pallas as pl
from jax.experimental.pallas import tpu as pltpu

N_DEV = 8
N_CHUNKS_H = 4


def kernel(x, w_mat):
    m, k_sh = x.shape
    _, n = w_mat.shape
    mb = m // N_DEV
    nh = n // 2
    nc = nh // N_CHUNKS_H

    def body(x_ref, w_ref, out_ref, comm_f, comm_r, amax_buf,
             send_f, recv_f, send_r, recv_r,
             amax_send_sems, amax_recv_sems,
             credit_f, credit_r, out_sems):
        p = lax.axis_index("i")
        left = lax.rem(p - 1 + N_DEV, N_DEV)
        right = lax.rem(p + 1, N_DEV)

        barrier_sem = pltpu.get_barrier_semaphore()
        for nbr in (left, right):
            pl.semaphore_signal(
                barrier_sem, 1,
                device_id=(nbr,), device_id_type=pl.DeviceIdType.MESH,
            )
        pl.semaphore_wait(barrier_sem, 2)

        def split_x(b):
            xs = x_ref[pl.ds(b * mb, mb), :]
            xs_hi = xs.astype(jnp.bfloat16)
            xs_lo = (xs - xs_hi.astype(jnp.float32)).astype(jnp.bfloat16)
            return xs_hi, xs_lo

        def dot3(xsplit, col_lo):
            xs_hi, xs_lo = xsplit
            wc = w_ref[:, pl.ds(col_lo, nc)]
            w_hi = wc.astype(jnp.bfloat16)
            w_lo = (wc - w_hi.astype(jnp.float32)).astype(jnp.bfloat16)
            acc = jnp.dot(xs_hi, w_hi, preferred_element_type=jnp.float32)
            acc += jnp.dot(xs_hi, w_lo, preferred_element_type=jnp.float32)
            acc += jnp.dot(xs_lo, w_hi, preferred_element_type=jnp.float32)
            return acc

        xf = split_x(lax.rem(p - 1 + N_DEV, N_DEV))
        xr = split_x(lax.rem(p + 1, N_DEV))
        for j in range(N_CHUNKS_H):
            cj = pl.ds(j * nc, nc)
            comm_f[0, :, cj] = dot3(xf, j * nc)
            comm_r[0, :, cj] = dot3(xr, nh + j * nc)

        for s in range(N_DEV - 1):
            ss, ns = s % 2, (s + 1) % 2
            last = s == N_DEV - 2
            if s >= 1:
                pl.semaphore_wait(credit_f, 1)
                pl.semaphore_wait(credit_r, 1)
            rds = []
            for j in range(N_CHUNKS_H):
                cj = pl.ds(j * nc, nc)
                rf = pltpu.make_async_remote_copy(
                    src_ref=comm_f.at[ss, :, cj],
                    dst_ref=comm_f.at[ns, :, cj],
                    send_sem=send_f.at[ss, j],
                    recv_sem=recv_f.at[ns, j],
                    device_id=(right,),
                    device_id_type=pl.DeviceIdType.MESH,
                )
                rr = pltpu.make_async_remote_copy(
                    src_ref=comm_r.at[ss, :, cj],
                    dst_ref=comm_r.at[ns, :, cj],
                    send_sem=send_r.at[ss, j],
                    recv_sem=recv_r.at[ns, j],
                    device_id=(left,),
                    device_id_type=pl.DeviceIdType.MESH,
                )
                rf.start()
                rr.start()
                rds.append((rf, rr))
            bf = lax.rem(p - s - 2 + 2 * N_DEV, N_DEV)
            br = lax.rem(p + s + 2, N_DEV)
            xf = split_x(bf)
            xr = split_x(br)
            dst = 0 if last else ns
            src = 1 if last else ns
            for j in range(N_CHUNKS_H):
                cj = pl.ds(j * nc, nc)
                rf, rr = rds[j]
                pbf = dot3(xf, j * nc)
                if last:
                    rf.wait_send()
                rf.wait_recv()
                comm_f[dst, :, cj] = comm_f[src, :, cj] + pbf
                pbr = dot3(xr, nh + j * nc)
                if last:
                    rr.wait_send()
                rr.wait_recv()
                comm_r[dst, :, cj] = comm_r[src, :, cj] + pbr
            if not last:
                for rf, rr in rds:
                    rf.wait_send()
                    rr.wait_send()
                pl.semaphore_signal(
                    credit_f, 1,
                    device_id=(left,), device_id_type=pl.DeviceIdType.MESH,
                )
                pl.semaphore_signal(
                    credit_r, 1,
                    device_id=(right,), device_id_type=pl.DeviceIdType.MESH,
                )

        amax_local = jnp.float32(0.0)
        for comm in (comm_f, comm_r):
            for j in range(N_CHUNKS_H):
                cj = pl.ds(j * nc, nc)
                amax_local = jnp.maximum(
                    amax_local, jnp.max(jnp.abs(comm[0, :, cj]))
                )
        amax_buf[pl.ds(0, 1), :] = jnp.full((1, 128), amax_local, jnp.float32)

        rds = []
        for d in range(1, N_DEV):
            tgt = lax.rem(p + d, N_DEV)
            rd = pltpu.make_async_remote_copy(
                src_ref=amax_buf.at[pl.ds(0, 1)],
                dst_ref=amax_buf.at[pl.ds(d, 1)],
                send_sem=amax_send_sems.at[d],
                recv_sem=amax_recv_sems.at[d],
                device_id=(tgt,),
                device_id_type=pl.DeviceIdType.MESH,
            )
            rd.start()
            rds.append(rd)
        for rd in rds:
            rd.wait_send()
        for rd in rds:
            rd.wait_recv()
        gmax = jnp.max(amax_buf[:, :])

        scale = gmax / 448.0
        inv_scale = 448.0 / gmax
        cps = []
        for comm, half in ((comm_f, 0), (comm_r, 1)):
            for j in range(N_CHUNKS_H):
                cj = pl.ds(j * nc, nc)
                q = jnp.clip(
                    comm[0, :, cj] * inv_scale, -448.0, 448.0
                ).astype(jnp.float8_e4m3fn)
                comm[0, :, cj] = q.astype(jnp.float32) * scale
                cp = pltpu.make_async_copy(
                    comm.at[0, :, cj],
                    out_ref.at[:, pl.ds(half * nh + j * nc, nc)],
                    out_sems.at[half * N_CHUNKS_H + j],
                )
                cp.start()
                cps.append(cp)
        for cp in cps:
            cp.wait()

    return pl.pallas_call(
        body,
        out_shape=jax.ShapeDtypeStruct((mb, n), jnp.float32),
        in_specs=[
            pl.BlockSpec(memory_space=pltpu.MemorySpace.VMEM),
            pl.BlockSpec(memory_space=pltpu.MemorySpace.VMEM),
        ],
        out_specs=pl.BlockSpec(memory_space=pltpu.MemorySpace.HBM),
        scratch_shapes=[
            pltpu.VMEM((2, mb, nh), jnp.float32),
            pltpu.VMEM((2, mb, nh), jnp.float32),
            pltpu.VMEM((N_DEV, 128), jnp.float32),
            pltpu.SemaphoreType.DMA((2, N_CHUNKS_H)),
            pltpu.SemaphoreType.DMA((2, N_CHUNKS_H)),
            pltpu.SemaphoreType.DMA((2, N_CHUNKS_H)),
            pltpu.SemaphoreType.DMA((2, N_CHUNKS_H)),
            pltpu.SemaphoreType.DMA((N_DEV,)),
            pltpu.SemaphoreType.DMA((N_DEV,)),
            pltpu.SemaphoreType.REGULAR,
            pltpu.SemaphoreType.REGULAR,
            pltpu.SemaphoreType.DMA((2 * N_CHUNKS_H,)),
        ],
        compiler_params=pltpu.CompilerParams(
            collective_id=0,
            vmem_limit_bytes=100 * 1024 * 1024,
        ),
    )(x, w_mat)


# device time: 670282 ns/iter; 2.1245x vs baseline; 1.0686x over previous
import jax
import jax.numpy as jnp
from jax import lax
from jax.experimental import pallas as pl
from jax.experimental.pallas import tpu as pltpu

N_DEV = 8
N_CHUNKS_H = 4


def kernel(x, w_mat):
    m, k_sh = x.shape
    _, n = w_mat.shape
    mb = m // N_DEV
    nh = n // 2
    nc = nh // N_CHUNKS_H

    def body(x_ref, w_ref, out_ref, comm_f, comm_r, amax_buf,
             send_f, recv_f, send_r, recv_r,
             amax_send_sems, amax_recv_sems,
             credit_f, credit_r, out_sems):
        p = lax.axis_index("i")
        left = lax.rem(p - 1 + N_DEV, N_DEV)
        right = lax.rem(p + 1, N_DEV)

        barrier_sem = pltpu.get_barrier_semaphore()
        for nbr in (left, right):
            pl.semaphore_signal(
                barrier_sem, 1,
                device_id=(nbr,), device_id_type=pl.DeviceIdType.MESH,
            )
        pl.semaphore_wait(barrier_sem, 2)

        def split_x(b):
            xs = x_ref[pl.ds(b * mb, mb), :]
            xs_hi = xs.astype(jnp.bfloat16)
            xs_lo = (xs - xs_hi.astype(jnp.float32)).astype(jnp.bfloat16)
            return xs_hi, xs_lo

        def dot3(xsplit, col_lo):
            xs_hi, xs_lo = xsplit
            wc = w_ref[:, pl.ds(col_lo, nc)]
            w_hi = wc.astype(jnp.bfloat16)
            w_lo = (wc - w_hi.astype(jnp.float32)).astype(jnp.bfloat16)
            acc = jnp.dot(xs_hi, w_hi, preferred_element_type=jnp.float32)
            acc += jnp.dot(xs_hi, w_lo, preferred_element_type=jnp.float32)
            acc += jnp.dot(xs_lo, w_hi, preferred_element_type=jnp.float32)
            return acc

        def mk_send(comm, sems_s, sems_r, s, j, dev):
            cj = pl.ds(j * nc, nc)
            return pltpu.make_async_remote_copy(
                src_ref=comm.at[s % 2, :, cj],
                dst_ref=comm.at[(s + 1) % 2, :, cj],
                send_sem=sems_s.at[s % 2, j],
                recv_sem=sems_r.at[(s + 1) % 2, j],
                device_id=(dev,),
                device_id_type=pl.DeviceIdType.MESH,
            )

        xf = split_x(lax.rem(p - 1 + N_DEV, N_DEV))
        xr = split_x(lax.rem(p + 1, N_DEV))
        cur_f, cur_r = [], []
        for j in range(N_CHUNKS_H):
            cj = pl.ds(j * nc, nc)
            comm_f[0, :, cj] = dot3(xf, j * nc)
            rf = mk_send(comm_f, send_f, recv_f, 0, j, right)
            rf.start()
            cur_f.append(rf)
            comm_r[0, :, cj] = dot3(xr, nh + j * nc)
            rr = mk_send(comm_r, send_r, recv_r, 0, j, left)
            rr.start()
            cur_r.append(rr)

        amax_local = jnp.float32(0.0)

        for s in range(N_DEV - 1):
            ns = (s + 1) % 2
            last = s == N_DEV - 2
            xf = split_x(lax.rem(p - s - 2 + 2 * N_DEV, N_DEV))
            xr = split_x(lax.rem(p + s + 2, N_DEV))
            nxt_f, nxt_r = [], []
            for j in range(N_CHUNKS_H):
                cj = pl.ds(j * nc, nc)
                for (comm, cur, nxt, sems_s, sems_r, credit, up, down,
                     col0) in (
                    (comm_f, cur_f, nxt_f, send_f, recv_f, credit_f,
                     left, right, 0),
                    (comm_r, cur_r, nxt_r, send_r, recv_r, credit_r,
                     right, left, nh),
                ):
                    pb = dot3(xf if col0 == 0 else xr, col0 + j * nc)
                    rd = cur[j]
                    rd.wait_recv()
                    if last:
                        rd.wait_send()
                        yc = comm[1, :, cj] + pb
                        comm[0, :, cj] = yc
                        amax_local = jnp.maximum(
                            amax_local, jnp.max(jnp.abs(yc))
                        )
                    else:
                        comm[ns, :, cj] = comm[ns, :, cj] + pb
                        rd.wait_send()
                        pl.semaphore_signal(
                            credit.at[j], 1,
                            device_id=(up,),
                            device_id_type=pl.DeviceIdType.MESH,
                        )
                        pl.semaphore_wait(credit.at[j], 1)
                        nrd = mk_send(comm, sems_s, sems_r, s + 1, j, down)
                        nrd.start()
                        nxt.append(nrd)
            cur_f, cur_r = nxt_f, nxt_r

        amax_buf[pl.ds(0, 1), :] = jnp.full((1, 128), amax_local, jnp.float32)

        rds = []
        for d in range(1, N_DEV):
            tgt = lax.rem(p + d, N_DEV)
            rd = pltpu.make_async_remote_copy(
                src_ref=amax_buf.at[pl.ds(0, 1)],
                dst_ref=amax_buf.at[pl.ds(d, 1)],
                send_sem=amax_send_sems.at[d],
                recv_sem=amax_recv_sems.at[d],
                device_id=(tgt,),
                device_id_type=pl.DeviceIdType.MESH,
            )
            rd.start()
            rds.append(rd)
        for rd in rds:
            rd.wait_send()
        for rd in rds:
            rd.wait_recv()
        gmax = jnp.max(amax_buf[:, :])

        scale = gmax / 448.0
        inv_scale = 448.0 / gmax
        cps = []
        for comm, half in ((comm_f, 0), (comm_r, 1)):
            for j in range(N_CHUNKS_H):
                cj = pl.ds(j * nc, nc)
                q = jnp.clip(
                    comm[0, :, cj] * inv_scale, -448.0, 448.0
                ).astype(jnp.float8_e4m3fn)
                comm[0, :, cj] = q.astype(jnp.float32) * scale
                cp = pltpu.make_async_copy(
                    comm.at[0, :, cj],
                    out_ref.at[:, pl.ds(half * nh + j * nc, nc)],
                    out_sems.at[half * N_CHUNKS_H + j],
                )
                cp.start()
                cps.append(cp)
        for cp in cps:
            cp.wait()

    return pl.pallas_call(
        body,
        out_shape=jax.ShapeDtypeStruct((mb, n), jnp.float32),
        in_specs=[
            pl.BlockSpec(memory_space=pltpu.MemorySpace.VMEM),
            pl.BlockSpec(memory_space=pltpu.MemorySpace.VMEM),
        ],
        out_specs=pl.BlockSpec(memory_space=pltpu.MemorySpace.HBM),
        scratch_shapes=[
            pltpu.VMEM((2, mb, nh), jnp.float32),
            pltpu.VMEM((2, mb, nh), jnp.float32),
            pltpu.VMEM((N_DEV, 128), jnp.float32),
            pltpu.SemaphoreType.DMA((2, N_CHUNKS_H)),
            pltpu.SemaphoreType.DMA((2, N_CHUNKS_H)),
            pltpu.SemaphoreType.DMA((2, N_CHUNKS_H)),
            pltpu.SemaphoreType.DMA((2, N_CHUNKS_H)),
            pltpu.SemaphoreType.DMA((N_DEV,)),
            pltpu.SemaphoreType.DMA((N_DEV,)),
            pltpu.SemaphoreType.REGULAR((N_CHUNKS_H,)),
            pltpu.SemaphoreType.REGULAR((N_CHUNKS_H,)),
            pltpu.SemaphoreType.DMA((2 * N_CHUNKS_H,)),
        ],
        compiler_params=pltpu.CompilerParams(
            collective_id=0,
            vmem_limit_bytes=100 * 1024 * 1024,
        ),
    )(x, w_mat)


# device time: 669368 ns/iter; 2.1274x vs baseline; 1.0014x over previous
import jax
import jax.numpy as jnp
from jax import lax
from jax.experimental import pallas as pl
from jax.experimental.pallas import tpu as pltpu

N_DEV = 8
N_CHUNKS_H = 8


def kernel(x, w_mat):
    m, k_sh = x.shape
    _, n = w_mat.shape
    mb = m // N_DEV
    nh = n // 2
    nc = nh // N_CHUNKS_H

    def body(x_ref, w_ref, out_ref, comm_f, comm_r, amax_buf,
             send_f, recv_f, send_r, recv_r,
             amax_send_sems, amax_recv_sems,
             credit_f, credit_r, out_sems):
        p = lax.axis_index("i")
        left = lax.rem(p - 1 + N_DEV, N_DEV)
        right = lax.rem(p + 1, N_DEV)

        barrier_sem = pltpu.get_barrier_semaphore()
        for nbr in (left, right):
            pl.semaphore_signal(
                barrier_sem, 1,
                device_id=(nbr,), device_id_type=pl.DeviceIdType.MESH,
            )
        pl.semaphore_wait(barrier_sem, 2)

        def split_x(b):
            xs = x_ref[pl.ds(b * mb, mb), :]
            xs_hi = xs.astype(jnp.bfloat16)
            xs_lo = (xs - xs_hi.astype(jnp.float32)).astype(jnp.bfloat16)
            return xs_hi, xs_lo

        def dot3(xsplit, col_lo):
            xs_hi, xs_lo = xsplit
            wc = w_ref[:, pl.ds(col_lo, nc)]
            w_hi = wc.astype(jnp.bfloat16)
            w_lo = (wc - w_hi.astype(jnp.float32)).astype(jnp.bfloat16)
            acc = jnp.dot(xs_hi, w_hi, preferred_element_type=jnp.float32)
            acc += jnp.dot(xs_hi, w_lo, preferred_element_type=jnp.float32)
            acc += jnp.dot(xs_lo, w_hi, preferred_element_type=jnp.float32)
            return acc

        def mk_send(comm, sems_s, sems_r, s, j, dev):
            cj = pl.ds(j * nc, nc)
            return pltpu.make_async_remote_copy(
                src_ref=comm.at[s % 2, :, cj],
                dst_ref=comm.at[(s + 1) % 2, :, cj],
                send_sem=sems_s.at[s % 2, j],
                recv_sem=sems_r.at[(s + 1) % 2, j],
                device_id=(dev,),
                device_id_type=pl.DeviceIdType.MESH,
            )

        xf = split_x(lax.rem(p - 1 + N_DEV, N_DEV))
        xr = split_x(lax.rem(p + 1, N_DEV))
        cur_f, cur_r = [], []
        for j in range(N_CHUNKS_H):
            cj = pl.ds(j * nc, nc)
            comm_f[0, :, cj] = dot3(xf, j * nc)
            rf = mk_send(comm_f, send_f, recv_f, 0, j, right)
            rf.start()
            cur_f.append(rf)
            comm_r[0, :, cj] = dot3(xr, nh + j * nc)
            rr = mk_send(comm_r, send_r, recv_r, 0, j, left)
            rr.start()
            cur_r.append(rr)

        amax_local = jnp.float32(0.0)

        for s in range(N_DEV - 1):
            ns = (s + 1) % 2
            last = s == N_DEV - 2
            xf = split_x(lax.rem(p - s - 2 + 2 * N_DEV, N_DEV))
            xr = split_x(lax.rem(p + s + 2, N_DEV))
            nxt_f, nxt_r = [], []
            for j in range(N_CHUNKS_H):
                cj = pl.ds(j * nc, nc)
                for (comm, cur, nxt, sems_s, sems_r, credit, up, down,
                     col0) in (
                    (comm_f, cur_f, nxt_f, send_f, recv_f, credit_f,
                     left, right, 0),
                    (comm_r, cur_r, nxt_r, send_r, recv_r, credit_r,
                     right, left, nh),
                ):
                    pb = dot3(xf if col0 == 0 else xr, col0 + j * nc)
                    rd = cur[j]
                    rd.wait_recv()
                    if last:
                        rd.wait_send()
                        yc = comm[1, :, cj] + pb
                        comm[0, :, cj] = yc
                        amax_local = jnp.maximum(
                            amax_local, jnp.max(jnp.abs(yc))
                        )
                    else:
                        comm[ns, :, cj] = comm[ns, :, cj] + pb
                        rd.wait_send()
                        pl.semaphore_signal(
                            credit.at[j], 1,
                            device_id=(up,),
                            device_id_type=pl.DeviceIdType.MESH,
                        )
                        pl.semaphore_wait(credit.at[j], 1)
                        nrd = mk_send(comm, sems_s, sems_r, s + 1, j, down)
                        nrd.start()
                        nxt.append(nrd)
            cur_f, cur_r = nxt_f, nxt_r

        amax_buf[pl.ds(0, 1), :] = jnp.full((1, 128), amax_local, jnp.float32)

        rds = []
        for d in range(1, N_DEV):
            tgt = lax.rem(p + d, N_DEV)
            rd = pltpu.make_async_remote_copy(
                src_ref=amax_buf.at[pl.ds(0, 1)],
                dst_ref=amax_buf.at[pl.ds(d, 1)],
                send_sem=amax_send_sems.at[d],
                recv_sem=amax_recv_sems.at[d],
                device_id=(tgt,),
                device_id_type=pl.DeviceIdType.MESH,
            )
            rd.start()
            rds.append(rd)
        for rd in rds:
            rd.wait_send()
        for rd in rds:
            rd.wait_recv()
        gmax = jnp.max(amax_buf[:, :])

        scale = gmax / 448.0
        inv_scale = 448.0 / gmax
        cps = []
        for comm, half in ((comm_f, 0), (comm_r, 1)):
            for j in range(N_CHUNKS_H):
                cj = pl.ds(j * nc, nc)
                q = jnp.clip(
                    comm[0, :, cj] * inv_scale, -448.0, 448.0
                ).astype(jnp.float8_e4m3fn)
                comm[0, :, cj] = q.astype(jnp.float32) * scale
                cp = pltpu.make_async_copy(
                    comm.at[0, :, cj],
                    out_ref.at[:, pl.ds(half * nh + j * nc, nc)],
                    out_sems.at[half * N_CHUNKS_H + j],
                )
                cp.start()
                cps.append(cp)
        for cp in cps:
            cp.wait()

    return pl.pallas_call(
        body,
        out_shape=jax.ShapeDtypeStruct((mb, n), jnp.float32),
        in_specs=[
            pl.BlockSpec(memory_space=pltpu.MemorySpace.VMEM),
            pl.BlockSpec(memory_space=pltpu.MemorySpace.VMEM),
        ],
        out_specs=pl.BlockSpec(memory_space=pltpu.MemorySpace.HBM),
        scratch_shapes=[
            pltpu.VMEM((2, mb, nh), jnp.float32),
            pltpu.VMEM((2, mb, nh), jnp.float32),
            pltpu.VMEM((N_DEV, 128), jnp.float32),
            pltpu.SemaphoreType.DMA((2, N_CHUNKS_H)),
            pltpu.SemaphoreType.DMA((2, N_CHUNKS_H)),
            pltpu.SemaphoreType.DMA((2, N_CHUNKS_H)),
            pltpu.SemaphoreType.DMA((2, N_CHUNKS_H)),
            pltpu.SemaphoreType.DMA((N_DEV,)),
            pltpu.SemaphoreType.DMA((N_DEV,)),
            pltpu.SemaphoreType.REGULAR((N_CHUNKS_H,)),
            pltpu.SemaphoreType.REGULAR((N_CHUNKS_H,)),
            pltpu.SemaphoreType.DMA((2 * N_CHUNKS_H,)),
        ],
        compiler_params=pltpu.CompilerParams(
            collective_id=0,
            vmem_limit_bytes=100 * 1024 * 1024,
        ),
    )(x, w_mat)
